# Initial kernel scaffold; baseline (speedup 1.0000x reference)
import jax
import jax.numpy as jnp
from jax.experimental import pallas as pl


def kernel(inputs, unnormalized_widths, unnormalized_heights, unnormalized_derivatives):
    raise NotImplementedError("write your pallas kernel here")


# speedup vs baseline: 1038.5979x; 1038.5979x over previous
import functools

import jax
import jax.numpy as jnp
from jax.experimental import pallas as pl
from jax.experimental.pallas import tpu as pltpu

B = 4096
D = 512
NUM_BINS = 32
TAIL_BOUND = 3.0
MIN_BIN_WIDTH = 1e-3
MIN_BIN_HEIGHT = 1e-3
MIN_DERIVATIVE = 1e-3

BB = 256


def _cumsum_lanes(x):
    n = x.shape[-1]
    shift = 1
    while shift < n:
        pad = jnp.zeros(x.shape[:-1] + (shift,), x.dtype)
        x = x + jnp.concatenate([pad, x[..., :-shift]], axis=-1)
        shift *= 2
    return x


def _normalized_cum(unnorm, min_frac):
    m = jnp.max(unnorm, axis=-1, keepdims=True)
    e = jnp.exp(unnorm - m)
    w = e / jnp.sum(e, axis=-1, keepdims=True)
    w = min_frac + (1.0 - min_frac * NUM_BINS) * w
    cs = _cumsum_lanes(w)
    full = jnp.concatenate([jnp.zeros((D, 1), jnp.float32), cs], axis=-1)
    full = 2.0 * TAIL_BOUND * full - TAIL_BOUND
    col = jax.lax.broadcasted_iota(jnp.int32, full.shape, 1)
    full = jnp.where(col == 0, -TAIL_BOUND, full)
    full = jnp.where(col == NUM_BINS, TAIL_BOUND, full)
    return full


def _prep_body(uw_ref, uh_ref, ud_ref,
               edges_ref, w_ref, ch_ref, dl_ref, d0_ref, d1_ref):
    cw_full = _normalized_cum(uw_ref[...], MIN_BIN_WIDTH)
    ch_full = _normalized_cum(uh_ref[...], MIN_BIN_HEIGHT)
    widths = cw_full[:, 1:] - cw_full[:, :-1]
    heights = ch_full[:, 1:] - ch_full[:, :-1]
    delta = heights / widths
    sp = MIN_DERIVATIVE + jnp.log1p(jnp.exp(ud_ref[...]))
    ones = jnp.ones((D, 1), jnp.float32)
    d_full = jnp.concatenate([ones, sp, ones], axis=-1)
    edges_ref[...] = cw_full[:, :NUM_BINS].T
    w_ref[...] = widths.T
    ch_ref[...] = ch_full[:, :NUM_BINS].T
    dl_ref[...] = delta.T
    d0_ref[...] = d_full[:, :NUM_BINS].T
    d1_ref[...] = d_full[:, 1:].T


def _prep_tables(uw, uh, ud):
    outs = [jax.ShapeDtypeStruct((NUM_BINS, D), jnp.float32)] * 6
    return pl.pallas_call(
        _prep_body,
        out_shape=outs,
    )(uw, uh, ud)


def _main_body(x_ref, edges_ref, w_ref, ch_ref, dl_ref, d0_ref, d1_ref,
               out_ref, lad_ref):
    x = x_ref[...]
    inside = (x >= -TAIL_BOUND) & (x <= TAIL_BOUND)
    xc = jnp.clip(x, -TAIL_BOUND, TAIL_BOUND)

    edges = edges_ref[...]
    tabs = [edges, w_ref[...], ch_ref[...], dl_ref[...], d0_ref[...],
            d1_ref[...]]
    zero = jnp.zeros_like(x)
    accs = [zero + t[0] for t in tabs]
    for k in range(1, NUM_BINS):
        ind = jnp.where(xc >= edges[k], 1.0, 0.0)
        for i, t in enumerate(tabs):
            accs[i] = accs[i] + ind * (t[k] - t[k - 1])
    cw, w, chh, dl, da, db = accs

    theta = (xc - cw) / w
    t2 = theta * theta
    t1m = theta - t2
    num = (dl * w) * (dl * t2 + da * t1m)
    den = dl + (da + db - 2.0 * dl) * t1m
    inv = 1.0 / den
    out_s = chh + num * inv
    omt = 1.0 - theta
    dnum = (dl * dl) * (db * t2 + 2.0 * dl * t1m + da * omt * omt)
    lad_s = jnp.log(dnum) - 2.0 * jnp.log(den)
    out_ref[...] = jnp.where(inside, out_s, x)
    lad_ref[...] = jnp.sum(jnp.where(inside, lad_s, 0.0), axis=1)


def _run_main(x, tables):
    grid = (B // BB,)
    tab_spec = pl.BlockSpec((NUM_BINS, D), lambda i: (0, 0))
    return pl.pallas_call(
        _main_body,
        grid=grid,
        in_specs=[pl.BlockSpec((BB, D), lambda i: (i, 0))] + [tab_spec] * 6,
        out_specs=[pl.BlockSpec((BB, D), lambda i: (i, 0)),
                   pl.BlockSpec((BB,), lambda i: (i,))],
        out_shape=[jax.ShapeDtypeStruct((B, D), jnp.float32),
                   jax.ShapeDtypeStruct((B,), jnp.float32)],
    )(x, *tables)


@jax.jit
def kernel(inputs, unnormalized_widths, unnormalized_heights,
           unnormalized_derivatives):
    tables = _prep_tables(unnormalized_widths, unnormalized_heights,
                          unnormalized_derivatives)
    out, lad = _run_main(inputs, tables)
    return out, lad
